# baseline (speedup 1.0000x reference)
import functools

import jax
import jax.numpy as jnp
from jax import lax
from jax.experimental import pallas as pl
from jax.experimental.pallas import tpu as pltpu
from jax.experimental.pallas import tpu_sc as plsc

D = 16
B = 16384
F = 20
L = 16
NC, NS = 2, 16
NW = NC * NS
PW = B // NW
CHUNK_P = 64
NCHUNK = PW // CHUNK_P
CHUNK_R = CHUNK_P * F
SEG = 128
NSEG = CHUNK_R // SEG
NG = CHUNK_P // L


def _rsqrt(x):
    i = lax.bitcast_convert_type(x, jnp.int32)
    i = jnp.int32(0x5F3759DF) - (i >> 1)
    y = lax.bitcast_convert_type(i, jnp.float32)
    for _ in range(3):
        y = y * (1.5 - 0.5 * x * y * y)
    return y


def _encoder_body(idx_hbm, table_hbm, out_hbm, idx_v, rows_v, out_v, sem):
    wid = lax.axis_index("s") * NC + lax.axis_index("c")
    iota = lax.iota(jnp.int32, L)
    d_idx = [jnp.full((L,), d, jnp.int32) for d in range(D)]

    for c in range(NCHUNK):
        pltpu.sync_copy(idx_hbm.at[wid, c], idx_v)
        copies = [
            pltpu.async_copy(
                table_hbm.at[idx_v.at[s]],
                rows_v.at[pl.ds(s * SEG, SEG)],
                sem,
            )
            for s in range(NSEG)
        ]
        for cp in copies:
            cp.wait()

        def g_body(g, carry):
            def l_body(l, acc):
                r_vec = iota * F + (g * (L * F) + l)
                v = [plsc.load_gather(rows_v, [r_vec, d_idx[d]]) for d in range(D)]
                ss = v[0] * v[0]
                for d in range(1, D):
                    ss = ss + v[d] * v[d]
                rn = _rsqrt(ss)
                return tuple(acc[d] + v[d] * rn for d in range(D))

            acc0 = tuple(jnp.zeros((L,), jnp.float32) for _ in range(D))
            acc = lax.fori_loop(0, F, l_body, acc0)

            ss2 = acc[0] * acc[0]
            for d in range(1, D):
                ss2 = ss2 + acc[d] * acc[d]
            rn2 = _rsqrt(ss2)
            p_vec = iota + g * L
            for d in range(D):
                plsc.store_scatter(out_v, [p_vec, d_idx[d]], acc[d] * rn2)
            return carry

        lax.fori_loop(0, NG, g_body, 0)

        pltpu.sync_copy(
            out_v, out_hbm.at[pl.ds(wid * PW + c * CHUNK_P, CHUNK_P)]
        )


_encode = functools.partial(
    pl.kernel,
    out_type=jax.ShapeDtypeStruct((B, D), jnp.float32),
    mesh=plsc.VectorSubcoreMesh(core_axis_name="c", subcore_axis_name="s"),
    scratch_types=[
        pltpu.VMEM((NSEG, SEG), jnp.int32),
        pltpu.VMEM((CHUNK_R, D), jnp.float32),
        pltpu.VMEM((CHUNK_P, D), jnp.float32),
        pltpu.SemaphoreType.DMA,
    ],
    compiler_params=pltpu.CompilerParams(
        needs_layout_passes=False, use_tc_tiling_on_sc=False
    ),
)(_encoder_body)


@jax.jit
def kernel(indices, table):
    idx = indices.astype(jnp.int32).reshape(NW, NCHUNK, NSEG, SEG)
    return _encode(idx, table)


# speedup vs baseline: 1.1203x; 1.1203x over previous
import functools

import jax
import jax.numpy as jnp
from jax import lax
from jax.experimental import pallas as pl
from jax.experimental.pallas import tpu as pltpu
from jax.experimental.pallas import tpu_sc as plsc

VOCAB = 1000000
D = 16
B = 16384
F = 20
L = 16
NC, NS = 2, 16
NW = NC * NS
PW = B // NW
CHUNK_P = 64
NCHUNK = PW // CHUNK_P
CHUNK_R = CHUNK_P * F
SEG = 128
NSEG = CHUNK_R // SEG
NG = CHUNK_P // L

BLK_C = 1024
NBLK = (VOCAB // 128) // 8
BLK_PER_W_HI = -(-NBLK // NW)
NT = (VOCAB + 127) // 128
TAIL_T0 = NBLK * 8
TAIL = VOCAB - (NT - 1) * 128


def _rsqrt(x):
    i = lax.bitcast_convert_type(x, jnp.int32)
    i = jnp.int32(0x5F3759DF) - (i >> 1)
    y = lax.bitcast_convert_type(i, jnp.float32)
    for _ in range(3):
        y = y * (1.5 - 0.5 * x * y * y)
    return y


def _transpose_body(tab_t, out_hbm, in_a, in_b, out_a, out_b, sem_ia, sem_ib,
                    sem_oa, sem_ob):
    wid = lax.axis_index("s") * NC + lax.axis_index("c")
    iota = lax.iota(jnp.int32, L)
    cnt = jnp.where(wid < NBLK - (BLK_PER_W_HI - 1) * NW, BLK_PER_W_HI,
                    BLK_PER_W_HI - 1)

    in_bufs = (in_a, in_b)
    out_bufs = (out_a, out_b)
    in_sems = (sem_ia, sem_ib)
    out_sems = (sem_oa, sem_ob)

    def issue_in(kb):
        blk = wid + NW * kb
        dst = in_bufs[kb & 1]
        sem = in_sems[kb & 1]
        c0 = blk * BLK_C
        pltpu.async_copy(tab_t.at[pl.ds(0, 8), pl.ds(c0, BLK_C)],
                         dst.at[pl.ds(0, 8)], sem)
        pltpu.async_copy(tab_t.at[pl.ds(8, 8), pl.ds(c0, BLK_C)],
                         dst.at[pl.ds(8, 8)], sem)

    def wait_in(kb):
        for _ in range(2):
            pltpu.make_async_copy(
                tab_t.at[pl.ds(0, 8), pl.ds(0, BLK_C)],
                in_bufs[kb & 1].at[pl.ds(0, 8)], in_sems[kb & 1]).wait()

    def wait_out(kb):
        pltpu.make_async_copy(
            out_bufs[kb & 1], out_hbm.at[pl.ds(0, BLK_C * D)],
            out_sems[kb & 1]).wait()

    @pl.when(0 < cnt)
    def _():
        issue_in(0)

    for kb in range(BLK_PER_W_HI):
        @pl.when(kb + 1 < cnt)
        def _(kb=kb):
            issue_in(kb + 1)

        @pl.when(kb < cnt)
        def _(kb=kb):
            wait_in(kb)
            if kb >= 2:
                wait_out(kb - 2)
            in_v = in_bufs[kb & 1]
            out_v = out_bufs[kb & 1]

            def r_body(r, c2):
                v = plsc.load_gather(in_v, [iota, jnp.broadcast_to(r, (L,))])
                out_v[pl.ds(r * D, D)] = v
                return c2

            lax.fori_loop(0, BLK_C, r_body, 0)
            blk = wid + NW * kb
            pltpu.async_copy(out_v, out_hbm.at[pl.ds(blk * (BLK_C * D),
                                                     BLK_C * D)],
                             out_sems[kb & 1])

    wait_out(0)
    wait_out(1)

    @pl.when(wid == NW - 1)
    def _():
        def tail_body(t, carry):
            j = TAIL_T0 + t
            pltpu.sync_copy(tab_t.at[pl.ds(0, 8), pl.ds(j * 128, 128)],
                            in_a.at[pl.ds(0, 8), pl.ds(0, 128)])
            pltpu.sync_copy(tab_t.at[pl.ds(8, 8), pl.ds(j * 128, 128)],
                            in_a.at[pl.ds(8, 8), pl.ds(0, 128)])
            nr = jnp.where(j == NT - 1, TAIL, 128)

            def r_body(r, c2):
                v = plsc.load_gather(in_a, [iota, jnp.broadcast_to(r, (L,))])
                out_a[pl.ds(r * D, D)] = v
                return c2

            lax.fori_loop(0, nr, r_body, 0)

            @pl.when(j != NT - 1)
            def _():
                pltpu.sync_copy(out_a.at[pl.ds(0, 128 * D)],
                                out_hbm.at[pl.ds(j * (128 * D), 128 * D)])

            @pl.when(j == NT - 1)
            def _():
                pltpu.sync_copy(out_a.at[pl.ds(0, TAIL * D)],
                                out_hbm.at[pl.ds(j * (128 * D), TAIL * D)])

            return carry

        lax.fori_loop(0, NT - TAIL_T0, tail_body, 0)


_transpose_sc = functools.partial(
    pl.kernel,
    out_type=jax.ShapeDtypeStruct((VOCAB * D,), jnp.float32),
    mesh=plsc.VectorSubcoreMesh(core_axis_name="c", subcore_axis_name="s"),
    scratch_types=[
        pltpu.VMEM((D, BLK_C), jnp.float32),
        pltpu.VMEM((D, BLK_C), jnp.float32),
        pltpu.VMEM((BLK_C * D,), jnp.float32),
        pltpu.VMEM((BLK_C * D,), jnp.float32),
        pltpu.SemaphoreType.DMA,
        pltpu.SemaphoreType.DMA,
        pltpu.SemaphoreType.DMA,
        pltpu.SemaphoreType.DMA,
    ],
    compiler_params=pltpu.CompilerParams(
        needs_layout_passes=False, use_tc_tiling_on_sc=True
    ),
)(_transpose_body)


def _encoder_body(idx_hbm, table_hbm, out_hbm, idx_a, idx_b, rows_a, rows_b,
                  outv_a, outv_b, sem_ga, sem_gb, sem_oa, sem_ob):
    wid = lax.axis_index("s") * NC + lax.axis_index("c")
    iota = lax.iota(jnp.int32, L)
    d_idx = [jnp.full((L,), d, jnp.int32) for d in range(D)]

    idx_bufs = (idx_a, idx_b)
    rows_bufs = (rows_a, rows_b)
    out_bufs = (outv_a, outv_b)
    g_sems = (sem_ga, sem_gb)
    o_sems = (sem_oa, sem_ob)

    def stage(c):
        pltpu.sync_copy(idx_hbm.at[wid, c], idx_bufs[c & 1])
        for s in range(NSEG):
            pltpu.async_copy(table_hbm.at[idx_bufs[c & 1].at[s]],
                             rows_bufs[c & 1].at[pl.ds(s * SEG, SEG)],
                             g_sems[c & 1])

    def wait_gathers(c):
        for s in range(NSEG):
            pltpu.make_async_copy(table_hbm.at[idx_bufs[c & 1].at[s]],
                                  rows_bufs[c & 1].at[pl.ds(s * SEG, SEG)],
                                  g_sems[c & 1]).wait()

    def wait_out(c):
        pltpu.make_async_copy(out_bufs[c & 1],
                              out_hbm.at[pl.ds(0, CHUNK_P)],
                              o_sems[c & 1]).wait()

    stage(0)
    for c in range(NCHUNK):
        if c + 1 < NCHUNK:
            stage(c + 1)
        wait_gathers(c)
        if c >= 2:
            wait_out(c - 2)
        rows_v = rows_bufs[c & 1]
        out_v = out_bufs[c & 1]

        def g_body(g, carry, rows_v=rows_v, out_v=out_v):
            def l_body(l, acc):
                r_vec = iota * F + (g * (L * F) + l)
                v = [plsc.load_gather(rows_v, [r_vec, d_idx[d]])
                     for d in range(D)]
                ss = v[0] * v[0]
                for d in range(1, D):
                    ss = ss + v[d] * v[d]
                rn = _rsqrt(ss)
                return tuple(acc[d] + v[d] * rn for d in range(D))

            acc0 = tuple(jnp.zeros((L,), jnp.float32) for _ in range(D))
            acc = lax.fori_loop(0, F, l_body, acc0)

            ss2 = acc[0] * acc[0]
            for d in range(1, D):
                ss2 = ss2 + acc[d] * acc[d]
            rn2 = _rsqrt(ss2)
            p_vec = iota + g * L
            for d in range(D):
                plsc.store_scatter(out_v, [p_vec, d_idx[d]], acc[d] * rn2)
            return carry

        lax.fori_loop(0, NG, g_body, 0)
        pltpu.async_copy(out_v,
                         out_hbm.at[pl.ds(wid * PW + c * CHUNK_P, CHUNK_P)],
                         o_sems[c & 1])

    wait_out(0)
    wait_out(1)


_encode = functools.partial(
    pl.kernel,
    out_type=jax.ShapeDtypeStruct((B, D), jnp.float32),
    mesh=plsc.VectorSubcoreMesh(core_axis_name="c", subcore_axis_name="s"),
    scratch_types=[
        pltpu.VMEM((NSEG, SEG), jnp.int32),
        pltpu.VMEM((NSEG, SEG), jnp.int32),
        pltpu.VMEM((CHUNK_R, D), jnp.float32),
        pltpu.VMEM((CHUNK_R, D), jnp.float32),
        pltpu.VMEM((CHUNK_P, D), jnp.float32),
        pltpu.VMEM((CHUNK_P, D), jnp.float32),
        pltpu.SemaphoreType.DMA,
        pltpu.SemaphoreType.DMA,
        pltpu.SemaphoreType.DMA,
        pltpu.SemaphoreType.DMA,
    ],
    compiler_params=pltpu.CompilerParams(
        needs_layout_passes=False, use_tc_tiling_on_sc=False
    ),
)(_encoder_body)


@jax.jit
def kernel(indices, table):
    idx = indices.astype(jnp.int32).reshape(NW, NCHUNK, NSEG, SEG)
    table_rm = _transpose_sc(table.T)
    return _encode(idx, table_rm.reshape(VOCAB, D))
